# Initial kernel scaffold; baseline (speedup 1.0000x reference)
import jax
import jax.numpy as jnp
from jax.experimental import pallas as pl


def kernel(token_type_ids, position_ids, inputs_embeds, W_pos, b_pos, W_tok, b_tok, ln_gamma, ln_beta):
    raise NotImplementedError("write your pallas kernel here")


# speedup vs baseline: 4.2870x; 4.2870x over previous
import functools
import math

import jax
import jax.numpy as jnp
import numpy as np
from jax.experimental import pallas as pl
from jax.experimental.pallas import tpu as pltpu

D_MODEL = 1024
POS_MAX = 300
TYPE_VOCAB = 4
LN_EPS = 1e-12

POS_PAD = 384
TOK_PAD = 8
BLK = 512


def _make_pe_np(d_model, max_len):
    position = np.arange(max_len, dtype=np.float32)[:, None]
    div_term = np.exp(
        np.arange(0, d_model, 2, dtype=np.float32) * -(math.log(1000.0) / d_model)
    )
    pe = np.zeros((max_len, d_model), dtype=np.float32)
    pe[:, 0::2] = np.sin(position * div_term)
    pe[:, 1::2] = np.cos(position * div_term)
    return pe


_PE_POS = np.zeros((POS_PAD, D_MODEL), dtype=np.float32)
_PE_POS[:POS_MAX] = _make_pe_np(D_MODEL, POS_MAX)
_PE_TOK = np.zeros((TOK_PAD, D_MODEL), dtype=np.float32)
_PE_TOK[:TYPE_VOCAB] = _make_pe_np(D_MODEL, TYPE_VOCAB)


def _proj_body(pe_pos_ref, pe_tok_ref, wpt_ref, bp_ref, wtt_ref, bt_ref,
               a_ref, tk_ref):
    a_ref[...] = (
        jnp.dot(pe_pos_ref[...], wpt_ref[...], preferred_element_type=jnp.float32)
        + bp_ref[...]
    )
    tk_ref[...] = (
        jnp.dot(pe_tok_ref[...], wtt_ref[...], preferred_element_type=jnp.float32)
        + bt_ref[...]
    )


def _fused_body(x_ref, pid_ref, tid_ref, a_ref, tk_ref, g_ref, b_ref, o_ref):
    x = x_ref[...]
    pid = pid_ref[...]
    iota = jax.lax.broadcasted_iota(jnp.int32, (BLK, POS_PAD), 1)
    onehot = (iota == pid).astype(jnp.float32)
    x = x + jnp.dot(onehot, a_ref[...], preferred_element_type=jnp.float32)
    tid = tid_ref[...]
    for t in range(TYPE_VOCAB):
        x = x + jnp.where(tid == t, 1.0, 0.0) * tk_ref[t:t + 1, :]
    mean = jnp.mean(x, axis=1, keepdims=True)
    xc = x - mean
    var = jnp.mean(xc * xc, axis=1, keepdims=True)
    o_ref[...] = xc * jax.lax.rsqrt(var + LN_EPS) * g_ref[...] + b_ref[...]


@functools.partial(jax.jit, static_argnames=("interpret",))
def kernel(token_type_ids, position_ids, inputs_embeds, W_pos, b_pos,
           W_tok, b_tok, ln_gamma, ln_beta, interpret=False):
    B, S, D = inputs_embeds.shape
    n = B * S
    x = inputs_embeds.reshape(n, D)
    pid = position_ids.reshape(n, 1).astype(jnp.int32)
    tid = token_type_ids.reshape(n, 1).astype(jnp.int32)

    a_tab, tk_tab = pl.pallas_call(
        _proj_body,
        out_shape=(
            jax.ShapeDtypeStruct((POS_PAD, D_MODEL), jnp.float32),
            jax.ShapeDtypeStruct((TOK_PAD, D_MODEL), jnp.float32),
        ),
        interpret=interpret,
    )(
        jnp.asarray(_PE_POS), jnp.asarray(_PE_TOK),
        W_pos.T, b_pos.reshape(1, D_MODEL),
        W_tok.T, b_tok.reshape(1, D_MODEL),
    )

    grid = (n // BLK,)
    out = pl.pallas_call(
        _fused_body,
        grid=grid,
        in_specs=[
            pl.BlockSpec((BLK, D_MODEL), lambda i: (i, 0)),
            pl.BlockSpec((BLK, 1), lambda i: (i, 0)),
            pl.BlockSpec((BLK, 1), lambda i: (i, 0)),
            pl.BlockSpec((POS_PAD, D_MODEL), lambda i: (0, 0)),
            pl.BlockSpec((TOK_PAD, D_MODEL), lambda i: (0, 0)),
            pl.BlockSpec((1, D_MODEL), lambda i: (0, 0)),
            pl.BlockSpec((1, D_MODEL), lambda i: (0, 0)),
        ],
        out_specs=pl.BlockSpec((BLK, D_MODEL), lambda i: (i, 0)),
        out_shape=jax.ShapeDtypeStruct((n, D_MODEL), jnp.float32),
        interpret=interpret,
    )(
        x, pid, tid, a_tab, tk_tab,
        ln_gamma.reshape(1, D_MODEL), ln_beta.reshape(1, D_MODEL),
    )
    return out.reshape(B, S, D)


# speedup vs baseline: 4.6447x; 1.0834x over previous
import functools
import math

import jax
import jax.numpy as jnp
import numpy as np
from jax.experimental import pallas as pl
from jax.experimental.pallas import tpu as pltpu

D_MODEL = 1024
POS_MAX = 300
TYPE_VOCAB = 4
LN_EPS = 1e-12

POS_PAD = 384
TOK_OFF = 384
TAB = 512
BLK = 512


def _make_pe_np(d_model, max_len):
    position = np.arange(max_len, dtype=np.float32)[:, None]
    div_term = np.exp(
        np.arange(0, d_model, 2, dtype=np.float32) * -(math.log(1000.0) / d_model)
    )
    pe = np.zeros((max_len, d_model), dtype=np.float32)
    pe[:, 0::2] = np.sin(position * div_term)
    pe[:, 1::2] = np.cos(position * div_term)
    return pe


_PE_POS = np.zeros((POS_PAD, D_MODEL), dtype=np.float32)
_PE_POS[:POS_MAX] = _make_pe_np(D_MODEL, POS_MAX)
_PE_TOK = np.zeros((8, D_MODEL), dtype=np.float32)
_PE_TOK[:TYPE_VOCAB] = _make_pe_np(D_MODEL, TYPE_VOCAB)


def _proj_body(pe_pos_ref, pe_tok_ref, wpt_ref, bp_ref, wtt_ref, bt_ref, t_ref):
    t_ref[0:POS_PAD, :] = (
        jnp.dot(pe_pos_ref[...], wpt_ref[...], preferred_element_type=jnp.float32)
        + bp_ref[...]
    )
    t_ref[TOK_OFF:TOK_OFF + 8, :] = (
        jnp.dot(pe_tok_ref[...], wtt_ref[...], preferred_element_type=jnp.float32)
        + bt_ref[...]
    )
    t_ref[TOK_OFF + 8:TAB, :] = jnp.zeros((TAB - TOK_OFF - 8, D_MODEL), jnp.float32)


def _fused_body(x_ref, pid_ref, tid_ref, t_ref, g_ref, b_ref, o_ref):
    x = x_ref[...]
    pid = pid_ref[...]
    tid = tid_ref[...]
    iota = jax.lax.broadcasted_iota(jnp.int32, (BLK, TAB), 1)
    sel = ((iota == pid) | (iota == tid + TOK_OFF)).astype(jnp.float32)
    x = x + jnp.dot(sel, t_ref[...], preferred_element_type=jnp.float32)
    mean = jnp.mean(x, axis=1, keepdims=True)
    xc = x - mean
    var = jnp.mean(xc * xc, axis=1, keepdims=True)
    o_ref[...] = xc * (jax.lax.rsqrt(var + LN_EPS) * g_ref[...]) + b_ref[...]


@functools.partial(jax.jit, static_argnames=("interpret",))
def kernel(token_type_ids, position_ids, inputs_embeds, W_pos, b_pos,
           W_tok, b_tok, ln_gamma, ln_beta, interpret=False):
    B, S, D = inputs_embeds.shape
    n = B * S
    x = inputs_embeds.reshape(n, D)
    pid = position_ids.reshape(n, 1).astype(jnp.int32)
    tid = token_type_ids.reshape(n, 1).astype(jnp.int32)

    tab = pl.pallas_call(
        _proj_body,
        out_shape=jax.ShapeDtypeStruct((TAB, D_MODEL), jnp.float32),
        interpret=interpret,
    )(
        jnp.asarray(_PE_POS), jnp.asarray(_PE_TOK),
        W_pos.T, b_pos.reshape(1, D_MODEL),
        W_tok.T, b_tok.reshape(1, D_MODEL),
    )

    grid = (n // BLK,)
    out = pl.pallas_call(
        _fused_body,
        grid=grid,
        in_specs=[
            pl.BlockSpec((BLK, D_MODEL), lambda i: (i, 0)),
            pl.BlockSpec((BLK, 1), lambda i: (i, 0)),
            pl.BlockSpec((BLK, 1), lambda i: (i, 0)),
            pl.BlockSpec((TAB, D_MODEL), lambda i: (0, 0)),
            pl.BlockSpec((1, D_MODEL), lambda i: (0, 0)),
            pl.BlockSpec((1, D_MODEL), lambda i: (0, 0)),
        ],
        out_specs=pl.BlockSpec((BLK, D_MODEL), lambda i: (i, 0)),
        out_shape=jax.ShapeDtypeStruct((n, D_MODEL), jnp.float32),
        interpret=interpret,
    )(
        x, pid, tid, tab,
        ln_gamma.reshape(1, D_MODEL), ln_beta.reshape(1, D_MODEL),
    )
    return out.reshape(B, S, D)


# speedup vs baseline: 5.1247x; 1.1034x over previous
import functools
import math

import jax
import jax.numpy as jnp
import numpy as np
from jax.experimental import pallas as pl
from jax.experimental.pallas import tpu as pltpu

D_MODEL = 1024
POS_MAX = 300
TYPE_VOCAB = 4
LN_EPS = 1e-12

POS_PAD = 384
TOK_OFF = 384
TAB = 512
BLK = 1024


def _make_pe_np(d_model, max_len):
    position = np.arange(max_len, dtype=np.float32)[:, None]
    div_term = np.exp(
        np.arange(0, d_model, 2, dtype=np.float32) * -(math.log(1000.0) / d_model)
    )
    pe = np.zeros((max_len, d_model), dtype=np.float32)
    pe[:, 0::2] = np.sin(position * div_term)
    pe[:, 1::2] = np.cos(position * div_term)
    return pe


_PE_POS = np.zeros((POS_PAD, D_MODEL), dtype=np.float32)
_PE_POS[:POS_MAX] = _make_pe_np(D_MODEL, POS_MAX)
_PE_TOK = np.zeros((8, D_MODEL), dtype=np.float32)
_PE_TOK[:TYPE_VOCAB] = _make_pe_np(D_MODEL, TYPE_VOCAB)


def _proj_body(pe_pos_ref, pe_tok_ref, wpt_ref, bp_ref, wtt_ref, bt_ref, t_ref):
    t_ref[0:POS_PAD, :] = (
        jnp.dot(pe_pos_ref[...], wpt_ref[...], preferred_element_type=jnp.float32)
        + bp_ref[...]
    )
    t_ref[TOK_OFF:TOK_OFF + 8, :] = (
        jnp.dot(pe_tok_ref[...], wtt_ref[...], preferred_element_type=jnp.float32)
        + bt_ref[...]
    )
    t_ref[TOK_OFF + 8:TAB, :] = jnp.zeros((TAB - TOK_OFF - 8, D_MODEL), jnp.float32)


def _fused_body(x_ref, pid_ref, tid_ref, t_ref, g_ref, b_ref, o_ref):
    x = x_ref[...]
    pid = pid_ref[...]
    tid = tid_ref[...]
    iota = jax.lax.broadcasted_iota(jnp.int32, (BLK, TAB), 1)
    sel = ((iota == pid) | (iota == tid + TOK_OFF)).astype(jnp.float32)
    x = x + jnp.dot(sel, t_ref[...], preferred_element_type=jnp.float32)
    mean = jnp.mean(x, axis=1, keepdims=True)
    xc = x - mean
    var = jnp.mean(xc * xc, axis=1, keepdims=True)
    o_ref[...] = xc * (jax.lax.rsqrt(var + LN_EPS) * g_ref[...]) + b_ref[...]


@functools.partial(jax.jit, static_argnames=("interpret",))
def kernel(token_type_ids, position_ids, inputs_embeds, W_pos, b_pos,
           W_tok, b_tok, ln_gamma, ln_beta, interpret=False):
    B, S, D = inputs_embeds.shape
    n = B * S
    x = inputs_embeds.reshape(n, D)
    pid = position_ids.reshape(n, 1).astype(jnp.int32)
    tid = token_type_ids.reshape(n, 1).astype(jnp.int32)

    tab = pl.pallas_call(
        _proj_body,
        out_shape=jax.ShapeDtypeStruct((TAB, D_MODEL), jnp.float32),
        interpret=interpret,
    )(
        jnp.asarray(_PE_POS), jnp.asarray(_PE_TOK),
        W_pos.T, b_pos.reshape(1, D_MODEL),
        W_tok.T, b_tok.reshape(1, D_MODEL),
    )

    grid = (n // BLK,)
    out = pl.pallas_call(
        _fused_body,
        grid=grid,
        in_specs=[
            pl.BlockSpec((BLK, D_MODEL), lambda i: (i, 0)),
            pl.BlockSpec((BLK, 1), lambda i: (i, 0)),
            pl.BlockSpec((BLK, 1), lambda i: (i, 0)),
            pl.BlockSpec((TAB, D_MODEL), lambda i: (0, 0)),
            pl.BlockSpec((1, D_MODEL), lambda i: (0, 0)),
            pl.BlockSpec((1, D_MODEL), lambda i: (0, 0)),
        ],
        out_specs=pl.BlockSpec((BLK, D_MODEL), lambda i: (i, 0)),
        out_shape=jax.ShapeDtypeStruct((n, D_MODEL), jnp.float32),
        interpret=interpret,
    )(
        x, pid, tid, tab,
        ln_gamma.reshape(1, D_MODEL), ln_beta.reshape(1, D_MODEL),
    )
    return out.reshape(B, S, D)


# speedup vs baseline: 5.2089x; 1.0164x over previous
import functools
import math

import jax
import jax.numpy as jnp
import numpy as np
from jax.experimental import pallas as pl
from jax.experimental.pallas import tpu as pltpu

D_MODEL = 1024
POS_MAX = 300
TYPE_VOCAB = 4
LN_EPS = 1e-12

POS_PAD = 384
TOK_OFF = 384
TAB = 512
BLK = 2048


def _make_pe_np(d_model, max_len):
    position = np.arange(max_len, dtype=np.float32)[:, None]
    div_term = np.exp(
        np.arange(0, d_model, 2, dtype=np.float32) * -(math.log(1000.0) / d_model)
    )
    pe = np.zeros((max_len, d_model), dtype=np.float32)
    pe[:, 0::2] = np.sin(position * div_term)
    pe[:, 1::2] = np.cos(position * div_term)
    return pe


_PE_POS = np.zeros((POS_PAD, D_MODEL), dtype=np.float32)
_PE_POS[:POS_MAX] = _make_pe_np(D_MODEL, POS_MAX)
_PE_TOK = np.zeros((8, D_MODEL), dtype=np.float32)
_PE_TOK[:TYPE_VOCAB] = _make_pe_np(D_MODEL, TYPE_VOCAB)


def _proj_body(pe_pos_ref, pe_tok_ref, wpt_ref, bp_ref, wtt_ref, bt_ref, t_ref):
    t_ref[0:POS_PAD, :] = (
        jnp.dot(pe_pos_ref[...], wpt_ref[...], preferred_element_type=jnp.float32)
        + bp_ref[...]
    )
    t_ref[TOK_OFF:TOK_OFF + 8, :] = (
        jnp.dot(pe_tok_ref[...], wtt_ref[...], preferred_element_type=jnp.float32)
        + bt_ref[...]
    )
    t_ref[TOK_OFF + 8:TAB, :] = jnp.zeros((TAB - TOK_OFF - 8, D_MODEL), jnp.float32)


def _fused_body(x_ref, pid_ref, tid_ref, t_ref, g_ref, b_ref, o_ref):
    x = x_ref[...]
    pid = pid_ref[...]
    tid = tid_ref[...]
    iota = jax.lax.broadcasted_iota(jnp.int32, (BLK, TAB), 1)
    sel = ((iota == pid) | (iota == tid + TOK_OFF)).astype(jnp.float32)
    x = x + jnp.dot(sel, t_ref[...], preferred_element_type=jnp.float32)
    mean = jnp.mean(x, axis=1, keepdims=True)
    xc = x - mean
    var = jnp.mean(xc * xc, axis=1, keepdims=True)
    o_ref[...] = xc * (jax.lax.rsqrt(var + LN_EPS) * g_ref[...]) + b_ref[...]


@functools.partial(jax.jit, static_argnames=("interpret",))
def kernel(token_type_ids, position_ids, inputs_embeds, W_pos, b_pos,
           W_tok, b_tok, ln_gamma, ln_beta, interpret=False):
    B, S, D = inputs_embeds.shape
    n = B * S
    x = inputs_embeds.reshape(n, D)
    pid = position_ids.reshape(n, 1).astype(jnp.int32)
    tid = token_type_ids.reshape(n, 1).astype(jnp.int32)

    tab = pl.pallas_call(
        _proj_body,
        out_shape=jax.ShapeDtypeStruct((TAB, D_MODEL), jnp.float32),
        interpret=interpret,
    )(
        jnp.asarray(_PE_POS), jnp.asarray(_PE_TOK),
        W_pos.T, b_pos.reshape(1, D_MODEL),
        W_tok.T, b_tok.reshape(1, D_MODEL),
    )

    grid = (n // BLK,)
    out = pl.pallas_call(
        _fused_body,
        grid=grid,
        in_specs=[
            pl.BlockSpec((BLK, D_MODEL), lambda i: (i, 0)),
            pl.BlockSpec((BLK, 1), lambda i: (i, 0)),
            pl.BlockSpec((BLK, 1), lambda i: (i, 0)),
            pl.BlockSpec((TAB, D_MODEL), lambda i: (0, 0)),
            pl.BlockSpec((1, D_MODEL), lambda i: (0, 0)),
            pl.BlockSpec((1, D_MODEL), lambda i: (0, 0)),
        ],
        out_specs=pl.BlockSpec((BLK, D_MODEL), lambda i: (i, 0)),
        out_shape=jax.ShapeDtypeStruct((n, D_MODEL), jnp.float32),
        interpret=interpret,
    )(
        x, pid, tid, tab,
        ln_gamma.reshape(1, D_MODEL), ln_beta.reshape(1, D_MODEL),
    )
    return out.reshape(B, S, D)
